# baseline (speedup 1.0000x reference)
import functools
import math

import jax
import jax.numpy as jnp
from jax.experimental import pallas as pl
from jax.experimental.pallas import tpu as pltpu

H = 128
NEG = -1e30


def _ln_mxu(x, g, b, J, eps=1e-5):
    m = jnp.dot(x, J, preferred_element_type=jnp.float32)
    xc = x - m
    v = jnp.dot(xc * xc, J, preferred_element_type=jnp.float32)
    return xc * jax.lax.rsqrt(v + eps) * g + b


def _fused_kernel(ids_a_ref, ids_b_ref, ids_c_ref,
                  pos_a_ref, pos_b_ref,
                  feat_b_ref, bfeat_ref,
                  centers_ref, inv2w2_ref, WgT_ref, bg_ref,
                  Wq_ref, bq_ref, Wkt_ref, Wkb_ref,
                  Wvt_ref, Wvb_ref,
                  Wc1_ref, bc1_ref, Wc2_ref, bc2_ref,
                  Wf1_ref, bf1_ref, Wf2_ref, bf2_ref,
                  ln1g_ref, ln1b_ref, ln2g_ref, ln2b_ref, J_ref,
                  out_ref,
                  psum_scr, cnt_scr, cent_scr, q_scr,
                  m_scr, s_scr, c_scr, h_scr, feat_scr,
                  *, nb, c, nsteps):
    i = pl.program_id(0)

    @pl.when(i == 0)
    def _init():
        psum_scr[...] = jnp.zeros((3, nb), jnp.float32)
        cnt_scr[...] = jnp.zeros((1, nb), jnp.float32)
        q_scr[...] = (jnp.dot(bfeat_ref[...], Wq_ref[...],
                              preferred_element_type=jnp.float32)
                      + bq_ref[...])
        m_scr[...] = jnp.full((nb, 1), NEG, jnp.float32)
        s_scr[...] = jnp.zeros((nb, 1), jnp.float32)
        c_scr[...] = jnp.zeros((nb, H), jnp.float32)

    @pl.when(i < nsteps)
    def _phase_a():
        idsv = ids_a_ref[0, 0, :]
        Of = (jax.lax.broadcasted_iota(jnp.int32, (nb, c), 0)
              == idsv[None, :]).astype(jnp.float32)
        psum_scr[...] += jax.lax.dot_general(
            pos_a_ref[...], Of, (((1,), (1,)), ((), ())),
            preferred_element_type=jnp.float32)
        cnt_scr[...] += jax.lax.dot_general(
            jnp.ones((1, c), jnp.float32), Of, (((1,), (1,)), ((), ())),
            preferred_element_type=jnp.float32)

        @pl.when(i == nsteps - 1)
        def _fin_a():
            cent_scr[...] = psum_scr[...] / jnp.maximum(cnt_scr[...], 1.0)

    @pl.when((i >= nsteps) & (i < 2 * nsteps))
    def _phase_b():
        idsv = ids_b_ref[0, 0, :]
        O = (jax.lax.broadcasted_iota(jnp.int32, (nb, c), 0)
             == idsv[None, :])
        Of = O.astype(jnp.float32)

        cent_g = jax.lax.dot_general(cent_scr[...], Of,
                                     (((1,), (0,)), ((), ())),
                                     preferred_element_type=jnp.float32)
        rel = pos_b_ref[...] - cent_g
        d = jnp.sqrt(jnp.sum(rel * rel, axis=0, keepdims=True))
        rbfT = jnp.exp(-jnp.square(d - centers_ref[...]) * inv2w2_ref[...])
        geomT = (jnp.dot(WgT_ref[...], rbfT,
                         preferred_element_type=jnp.float32)
                 + bg_ref[...]).astype(jnp.bfloat16)
        geom_aug = jnp.concatenate(
            [geomT, jnp.ones((1, c), jnp.bfloat16)], axis=0)

        feats_f32 = feat_b_ref[...]
        feat_scr[pl.ds((i - nsteps) * c, c), :] = feats_f32
        feats = feats_f32.astype(jnp.bfloat16)
        K = (jnp.dot(feats, Wkt_ref[...], preferred_element_type=jnp.float32)
             + jax.lax.dot_general(geom_aug, Wkb_ref[...],
                                   (((0,), (0,)), ((), ())),
                                   preferred_element_type=jnp.float32))
        V = (jnp.dot(feats, Wvt_ref[...],
                     preferred_element_type=jnp.float32)
             + jax.lax.dot_general(geom_aug, Wvb_ref[...],
                                   (((0,), (0,)), ((), ())),
                                   preferred_element_type=jnp.float32)
             ).astype(jnp.bfloat16)

        S = jax.lax.dot_general(q_scr[...], K, (((1,), (1,)), ((), ())),
                                preferred_element_type=jnp.float32)
        S = S * (1.0 / math.sqrt(H))
        Sm = jnp.where(O, S, NEG)
        m_old = m_scr[...]
        m_new = jnp.maximum(m_old, jnp.max(Sm, axis=1, keepdims=True))
        alpha = jnp.exp(m_old - m_new)
        e = jnp.where(O, jnp.exp(S - m_new), 0.0)
        m_scr[...] = m_new
        s_scr[...] = s_scr[...] * alpha + jnp.sum(e, axis=1, keepdims=True)
        c_scr[...] = (c_scr[...] * alpha
                      + jnp.dot(e.astype(jnp.bfloat16), V,
                                preferred_element_type=jnp.float32))

        @pl.when(i == 2 * nsteps - 1)
        def _fin_b():
            s = s_scr[...]
            ctx = c_scr[...] / jnp.where(s > 0.0, s, 1.0)
            h1 = jnp.maximum(
                jnp.dot(ctx, Wc1_ref[...],
                        preferred_element_type=jnp.float32)
                + bc1_ref[...], 0.0)
            h_scr[...] = (jnp.dot(h1, Wc2_ref[...],
                                  preferred_element_type=jnp.float32)
                          + bc2_ref[...])

    @pl.when(i >= 2 * nsteps)
    def _phase_c():
        idsv = ids_c_ref[0, 0, :]
        Of = (jax.lax.broadcasted_iota(jnp.int32, (nb, c), 0)
              == idsv[None, :]).astype(jnp.float32)
        upd = jax.lax.dot_general(Of, h_scr[...], (((0,), (0,)), ((), ())),
                                  preferred_element_type=jnp.float32)
        feats = feat_scr[pl.ds((i - 2 * nsteps) * c, c), :]
        u1 = _ln_mxu(feats + upd, ln1g_ref[...], ln1b_ref[...], J_ref[...])
        f1 = jnp.maximum(
            jnp.dot(u1.astype(jnp.bfloat16), Wf1_ref[...],
                    preferred_element_type=jnp.float32)
            + bf1_ref[...], 0.0)
        ffn = (jnp.dot(f1.astype(jnp.bfloat16), Wf2_ref[...],
                       preferred_element_type=jnp.float32)
               + bf2_ref[...])
        out_ref[...] = _ln_mxu(u1 + ffn, ln2g_ref[...], ln2b_ref[...],
                               J_ref[...])


def kernel(atom_features, atom_positions, block_features, block_id,
           centers, widths, Wg, bg, Wq, bq, Wk, bk, Wv, bv,
           Wc1, bc1, Wc2, bc2, Wf1, bf1, Wf2, bf2,
           ln1_g, ln1_b, ln2_g, ln2_b):
    n, h = atom_features.shape
    nb = block_features.shape[0]
    rbf_dim = centers.shape[0]
    hq = Wg.shape[1]
    C = 8192
    nsteps = n // C

    ids = block_id.astype(jnp.int32)
    ids_chunked = ids.reshape(nsteps, 1, C)
    posT = atom_positions.T

    centers_col = centers.reshape(rbf_dim, 1).astype(jnp.float32)
    inv2w2_col = (1.0 / (2.0 * jnp.square(widths))).reshape(rbf_dim, 1)
    row = lambda v: v.reshape(1, -1)

    Wk_top = Wk[:h].astype(jnp.bfloat16)
    Wv_top = Wv[:h].astype(jnp.bfloat16)
    Wk_bot = jnp.concatenate([Wk[h:], bk.reshape(1, h)], 0).astype(jnp.bfloat16)
    Wv_bot = jnp.concatenate([Wv[h:], bv.reshape(1, h)], 0).astype(jnp.bfloat16)
    Jmat = jnp.full((h, h), 1.0 / h, jnp.float32)
    Wf1_bf = Wf1.astype(jnp.bfloat16)
    Wf2_bf = Wf2.astype(jnp.bfloat16)
    WgT = Wg.T
    bg_col = bg.reshape(hq, 1)

    last = nsteps - 1
    chunk_a = lambda i: (jnp.clip(i, 0, last), 0, 0)
    chunk_b = lambda i: (jnp.clip(i - nsteps, 0, last), 0, 0)
    chunk_c = lambda i: (jnp.clip(i - 2 * nsteps, 0, last), 0, 0)
    posm_a = lambda i: (0, jnp.clip(i, 0, last))
    posm_b = lambda i: (0, jnp.clip(i - nsteps, 0, last))
    featm_b = lambda i: (jnp.clip(i - nsteps, 0, last), 0)
    featm_c = lambda i: (jnp.clip(i - 2 * nsteps, 0, last), 0)
    full = lambda shape: pl.BlockSpec(shape, lambda i: (0,) * len(shape))

    out = pl.pallas_call(
        functools.partial(_fused_kernel, nb=nb, c=C, nsteps=nsteps),
        grid=(3 * nsteps,),
        in_specs=[
            pl.BlockSpec((1, 1, C), chunk_a),
            pl.BlockSpec((1, 1, C), chunk_b),
            pl.BlockSpec((1, 1, C), chunk_c),
            pl.BlockSpec((3, C), posm_a),
            pl.BlockSpec((3, C), posm_b),
            pl.BlockSpec((C, h), featm_b),
            full((nb, h)),
            full((rbf_dim, 1)),
            full((rbf_dim, 1)),
            full((hq, rbf_dim)),
            full((hq, 1)),
            full((h, h)),
            full((1, h)),
            full((h, h)),
            full((hq + 1, h)),
            full((h, h)),
            full((hq + 1, h)),
            full((h, h)),
            full((1, h)),
            full((h, h)),
            full((1, h)),
            full((h, 2 * h)),
            full((1, 2 * h)),
            full((2 * h, h)),
            full((1, h)),
            full((1, h)),
            full((1, h)),
            full((1, h)),
            full((1, h)),
            full((h, h)),
        ],
        out_specs=pl.BlockSpec((C, h), featm_c),
        out_shape=jax.ShapeDtypeStruct((n, h), jnp.float32),
        scratch_shapes=[
            pltpu.VMEM((3, nb), jnp.float32),
            pltpu.VMEM((1, nb), jnp.float32),
            pltpu.VMEM((3, nb), jnp.float32),
            pltpu.VMEM((nb, h), jnp.float32),
            pltpu.VMEM((nb, 1), jnp.float32),
            pltpu.VMEM((nb, 1), jnp.float32),
            pltpu.VMEM((nb, h), jnp.float32),
            pltpu.VMEM((nb, h), jnp.float32),
            pltpu.VMEM((n, h), jnp.float32),
        ],
    )(ids_chunked, ids_chunked, ids_chunked,
      posT, posT,
      atom_features, block_features,
      centers_col, inv2w2_col, WgT, bg_col,
      Wq, row(bq), Wk_top, Wk_bot,
      Wv_top, Wv_bot,
      Wc1, row(bc1), Wc2, row(bc2),
      Wf1_bf, row(bf1), Wf2_bf, row(bf2),
      row(ln1_g), row(ln1_b), row(ln2_g), row(ln2_b), Jmat)

    return out


# speedup vs baseline: 1.0275x; 1.0275x over previous
import functools
import math

import jax
import jax.numpy as jnp
from jax.experimental import pallas as pl
from jax.experimental.pallas import tpu as pltpu

H = 128
NEG = -1e30


def _ln_mxu(x, g, b, J, eps=1e-5):
    m = jnp.dot(x, J, preferred_element_type=jnp.float32)
    xc = x - m
    v = jnp.dot(xc * xc, J, preferred_element_type=jnp.float32)
    return xc * jax.lax.rsqrt(v + eps) * g + b


def _fused_kernel(ids_a_ref, ids_b_ref, ids_c_ref,
                  pos_a_ref, pos_b_ref,
                  feat_b_ref, bfeat_ref,
                  centers_ref, inv2w2_ref, WgT_ref, bg_ref,
                  Wq_ref, bq_ref, Wkt_ref, Wkb_ref,
                  Wvt_ref, Wvb_ref,
                  Wc1_ref, bc1_ref, Wc2_ref, bc2_ref,
                  Wf1_ref, bf1_ref, Wf2_ref, bf2_ref,
                  ln1g_ref, ln1b_ref, ln2g_ref, ln2b_ref, J_ref,
                  out_ref,
                  psum_scr, cnt_scr, cent_scr, qk1_scr, qk2_scr,
                  m_scr, a1_scr, h_scr, feat_scr,
                  *, nb, c, nsteps):
    i = pl.program_id(0)

    @pl.when(i == 0)
    def _init():
        psum_scr[...] = jnp.zeros((3, nb), jnp.float32)
        cnt_scr[...] = jnp.zeros((1, nb), jnp.float32)
        q = (jnp.dot(bfeat_ref[...], Wq_ref[...],
                     preferred_element_type=jnp.float32)
             + bq_ref[...]) * (1.0 / math.sqrt(H))
        qk1_scr[...] = jax.lax.dot_general(
            q, Wkt_ref[...].astype(jnp.float32), (((1,), (1,)), ((), ())),
            preferred_element_type=jnp.float32).astype(jnp.bfloat16)
        qk2_scr[...] = jax.lax.dot_general(
            q, Wkb_ref[...].astype(jnp.float32), (((1,), (1,)), ((), ())),
            preferred_element_type=jnp.float32).astype(jnp.bfloat16)
        m_scr[...] = jnp.full((nb, 1), NEG, jnp.float32)
        a1_scr[...] = jnp.zeros(a1_scr.shape, jnp.float32)

    @pl.when(i < nsteps)
    def _phase_a():
        idsv = ids_a_ref[0, 0, :]
        Of = (jax.lax.broadcasted_iota(jnp.int32, (nb, c), 0)
              == idsv[None, :]).astype(jnp.float32)
        psum_scr[...] += jax.lax.dot_general(
            pos_a_ref[...], Of, (((1,), (1,)), ((), ())),
            preferred_element_type=jnp.float32)
        cnt_scr[...] += jax.lax.dot_general(
            jnp.ones((1, c), jnp.float32), Of, (((1,), (1,)), ((), ())),
            preferred_element_type=jnp.float32)

        @pl.when(i == nsteps - 1)
        def _fin_a():
            cent_scr[...] = psum_scr[...] / jnp.maximum(cnt_scr[...], 1.0)

    @pl.when((i >= nsteps) & (i < 2 * nsteps))
    def _phase_b():
        idsv = ids_b_ref[0, 0, :]
        O = (jax.lax.broadcasted_iota(jnp.int32, (nb, c), 0)
             == idsv[None, :])
        Of = O.astype(jnp.float32)

        cent_g = jax.lax.dot_general(cent_scr[...], Of,
                                     (((1,), (0,)), ((), ())),
                                     preferred_element_type=jnp.float32)
        rel = pos_b_ref[...] - cent_g
        d = jnp.sqrt(jnp.sum(rel * rel, axis=0, keepdims=True))
        rbfT = jnp.exp(-jnp.square(d - centers_ref[...]) * inv2w2_ref[...])
        geomT = (jnp.dot(WgT_ref[...], rbfT,
                         preferred_element_type=jnp.float32)
                 + bg_ref[...]).astype(jnp.bfloat16)

        feats_f32 = feat_b_ref[...]
        feat_scr[pl.ds((i - nsteps) * c, c), :] = feats_f32
        feats = feats_f32.astype(jnp.bfloat16)
        aug = jnp.concatenate(
            [feats, jnp.transpose(geomT, (1, 0)),
             jnp.ones((c, 1), jnp.bfloat16)], axis=1)
        qkc = jnp.concatenate([qk1_scr[...], qk2_scr[...]], axis=1)

        S = jax.lax.dot_general(qkc, aug, (((1,), (1,)), ((), ())),
                                preferred_element_type=jnp.float32)
        Sm = jnp.where(O, S, NEG)
        m_old = m_scr[...]
        m_new = jnp.maximum(m_old, jnp.max(Sm, axis=1, keepdims=True))
        alpha = jnp.exp(m_old - m_new)
        e = jnp.where(O, jnp.exp(S - m_new), 0.0)
        m_scr[...] = m_new
        a1_scr[...] = (a1_scr[...] * alpha
                       + jnp.dot(e.astype(jnp.bfloat16), aug,
                                 preferred_element_type=jnp.float32))

        @pl.when(i == 2 * nsteps - 1)
        def _fin_b():
            acc = a1_scr[...]
            s = acc[:, H + 32:H + 33]
            craw = (jnp.dot(acc[:, :H], Wvt_ref[...].astype(jnp.float32),
                            preferred_element_type=jnp.float32)
                    + jax.lax.dot_general(
                        acc[:, H:H + 33], Wvb_ref[...].astype(jnp.float32),
                        (((1,), (0,)), ((), ())),
                        preferred_element_type=jnp.float32))
            ctx = craw / jnp.where(s > 0.0, s, 1.0)
            h1 = jnp.maximum(
                jnp.dot(ctx, Wc1_ref[...],
                        preferred_element_type=jnp.float32)
                + bc1_ref[...], 0.0)
            h_scr[...] = (jnp.dot(h1, Wc2_ref[...],
                                  preferred_element_type=jnp.float32)
                          + bc2_ref[...])

    @pl.when(i >= 2 * nsteps)
    def _phase_c():
        idsv = ids_c_ref[0, 0, :]
        Of = (jax.lax.broadcasted_iota(jnp.int32, (nb, c), 0)
              == idsv[None, :]).astype(jnp.float32)
        upd = jax.lax.dot_general(Of, h_scr[...], (((0,), (0,)), ((), ())),
                                  preferred_element_type=jnp.float32)
        feats = feat_scr[pl.ds((i - 2 * nsteps) * c, c), :]
        u1 = _ln_mxu(feats + upd, ln1g_ref[...], ln1b_ref[...], J_ref[...])
        f1 = jnp.maximum(
            jnp.dot(u1.astype(jnp.bfloat16), Wf1_ref[...],
                    preferred_element_type=jnp.float32)
            + bf1_ref[...], 0.0)
        ffn = (jnp.dot(f1.astype(jnp.bfloat16), Wf2_ref[...],
                       preferred_element_type=jnp.float32)
               + bf2_ref[...])
        out_ref[...] = _ln_mxu(u1 + ffn, ln2g_ref[...], ln2b_ref[...],
                               J_ref[...])


def kernel(atom_features, atom_positions, block_features, block_id,
           centers, widths, Wg, bg, Wq, bq, Wk, bk, Wv, bv,
           Wc1, bc1, Wc2, bc2, Wf1, bf1, Wf2, bf2,
           ln1_g, ln1_b, ln2_g, ln2_b):
    n, h = atom_features.shape
    nb = block_features.shape[0]
    rbf_dim = centers.shape[0]
    hq = Wg.shape[1]
    C = 8192
    nsteps = n // C

    ids = block_id.astype(jnp.int32)
    ids_chunked = ids.reshape(nsteps, 1, C)
    posT = atom_positions.T

    centers_col = centers.reshape(rbf_dim, 1).astype(jnp.float32)
    inv2w2_col = (1.0 / (2.0 * jnp.square(widths))).reshape(rbf_dim, 1)
    row = lambda v: v.reshape(1, -1)

    Wk_top = Wk[:h].astype(jnp.bfloat16)
    Wv_top = Wv[:h].astype(jnp.bfloat16)
    Wk_bot = jnp.concatenate([Wk[h:], bk.reshape(1, h)], 0).astype(jnp.bfloat16)
    Wv_bot = jnp.concatenate([Wv[h:], bv.reshape(1, h)], 0).astype(jnp.bfloat16)
    Jmat = jnp.full((h, h), 1.0 / h, jnp.float32)
    Wf1_bf = Wf1.astype(jnp.bfloat16)
    Wf2_bf = Wf2.astype(jnp.bfloat16)
    WgT = Wg.T
    bg_col = bg.reshape(hq, 1)

    last = nsteps - 1
    chunk_a = lambda i: (jnp.clip(i, 0, last), 0, 0)
    chunk_b = lambda i: (jnp.clip(i - nsteps, 0, last), 0, 0)
    chunk_c = lambda i: (jnp.clip(i - 2 * nsteps, 0, last), 0, 0)
    posm_a = lambda i: (0, jnp.clip(i, 0, last))
    posm_b = lambda i: (0, jnp.clip(i - nsteps, 0, last))
    featm_b = lambda i: (jnp.clip(i - nsteps, 0, last), 0)
    featm_c = lambda i: (jnp.clip(i - 2 * nsteps, 0, last), 0)
    full = lambda shape: pl.BlockSpec(shape, lambda i: (0,) * len(shape))

    out = pl.pallas_call(
        functools.partial(_fused_kernel, nb=nb, c=C, nsteps=nsteps),
        grid=(3 * nsteps,),
        in_specs=[
            pl.BlockSpec((1, 1, C), chunk_a),
            pl.BlockSpec((1, 1, C), chunk_b),
            pl.BlockSpec((1, 1, C), chunk_c),
            pl.BlockSpec((3, C), posm_a),
            pl.BlockSpec((3, C), posm_b),
            pl.BlockSpec((C, h), featm_b),
            full((nb, h)),
            full((rbf_dim, 1)),
            full((rbf_dim, 1)),
            full((hq, rbf_dim)),
            full((hq, 1)),
            full((h, h)),
            full((1, h)),
            full((h, h)),
            full((hq + 1, h)),
            full((h, h)),
            full((hq + 1, h)),
            full((h, h)),
            full((1, h)),
            full((h, h)),
            full((1, h)),
            full((h, 2 * h)),
            full((1, 2 * h)),
            full((2 * h, h)),
            full((1, h)),
            full((1, h)),
            full((1, h)),
            full((1, h)),
            full((1, h)),
            full((h, h)),
        ],
        out_specs=pl.BlockSpec((C, h), featm_c),
        out_shape=jax.ShapeDtypeStruct((n, h), jnp.float32),
        scratch_shapes=[
            pltpu.VMEM((3, nb), jnp.float32),
            pltpu.VMEM((1, nb), jnp.float32),
            pltpu.VMEM((3, nb), jnp.float32),
            pltpu.VMEM((nb, h), jnp.bfloat16),
            pltpu.VMEM((nb, hq + 1), jnp.bfloat16),
            pltpu.VMEM((nb, 1), jnp.float32),
            pltpu.VMEM((nb, h + hq + 1), jnp.float32),
            pltpu.VMEM((nb, h), jnp.float32),
            pltpu.VMEM((n, h), jnp.float32),
        ],
    )(ids_chunked, ids_chunked, ids_chunked,
      posT, posT,
      atom_features, block_features,
      centers_col, inv2w2_col, WgT, bg_col,
      Wq, row(bq), Wk_top, Wk_bot,
      Wv_top, Wv_bot,
      Wc1, row(bc1), Wc2, row(bc2),
      Wf1_bf, row(bf1), Wf2_bf, row(bf2),
      row(ln1_g), row(ln1_b), row(ln2_g), row(ln2_b), Jmat)

    return out


# speedup vs baseline: 1.1477x; 1.1170x over previous
import functools
import math

import jax
import jax.numpy as jnp
from jax.experimental import pallas as pl
from jax.experimental.pallas import tpu as pltpu

H = 128
NEG = -1e30


def _ln_mxu(x, g, b, J, eps=1e-5):
    m = jnp.dot(x, J, preferred_element_type=jnp.float32)
    xc = x - m
    v = jnp.dot(xc * xc, J, preferred_element_type=jnp.float32)
    return xc * jax.lax.rsqrt(v + eps) * g + b


def _fused_kernel(ids_a_ref, ids_b_ref, ids_c_ref,
                  pos_a_ref, pos_b_ref,
                  feat_b_ref, bfeat_ref,
                  centers_ref, inv2w2_ref, WgT_ref, bg_ref,
                  Wq_ref, bq_ref, Wkt_ref, Wkb_ref,
                  Wvt_ref, Wvb_ref,
                  Wc1_ref, bc1_ref, Wc2_ref, bc2_ref,
                  Wf1_ref, bf1_ref, Wf2_ref, bf2_ref,
                  ln1g_ref, ln1b_ref, ln2g_ref, ln2b_ref, J_ref,
                  out_ref,
                  psum_scr, cnt_scr, cent_scr, qk1_scr, qk2_scr,
                  m_scr, s_scr, a1_scr, a2_scr, h_scr, feat_scr,
                  *, nb, c, nsteps):
    i = pl.program_id(0)

    @pl.when(i == 0)
    def _init():
        psum_scr[...] = jnp.zeros((3, nb), jnp.float32)
        cnt_scr[...] = jnp.zeros((1, nb), jnp.float32)
        q = (jnp.dot(bfeat_ref[...], Wq_ref[...],
                     preferred_element_type=jnp.float32)
             + bq_ref[...]) * (1.0 / math.sqrt(H))
        qk1_scr[...] = jax.lax.dot_general(
            q, Wkt_ref[...].astype(jnp.float32), (((1,), (1,)), ((), ())),
            preferred_element_type=jnp.float32).astype(jnp.bfloat16)
        qk2_scr[...] = jax.lax.dot_general(
            q, Wkb_ref[...].astype(jnp.float32), (((1,), (1,)), ((), ())),
            preferred_element_type=jnp.float32).astype(jnp.bfloat16)
        m_scr[...] = jnp.full((nb, 1), NEG, jnp.float32)
        s_scr[...] = jnp.zeros((nb, 1), jnp.float32)
        a1_scr[...] = jnp.zeros(a1_scr.shape, jnp.float32)
        a2_scr[...] = jnp.zeros(a2_scr.shape, jnp.float32)

    @pl.when(i < nsteps)
    def _phase_a():
        idsv = ids_a_ref[0, 0, :]
        Of = (jax.lax.broadcasted_iota(jnp.int32, (nb, c), 0)
              == idsv[None, :]).astype(jnp.float32)
        psum_scr[...] += jax.lax.dot_general(
            pos_a_ref[...], Of, (((1,), (1,)), ((), ())),
            preferred_element_type=jnp.float32)
        cnt_scr[...] += jax.lax.dot_general(
            jnp.ones((1, c), jnp.float32), Of, (((1,), (1,)), ((), ())),
            preferred_element_type=jnp.float32)

        @pl.when(i == nsteps - 1)
        def _fin_a():
            cent_scr[...] = psum_scr[...] / jnp.maximum(cnt_scr[...], 1.0)

    @pl.when((i >= nsteps) & (i < 2 * nsteps))
    def _phase_b():
        idsv = ids_b_ref[0, 0, :]
        O = (jax.lax.broadcasted_iota(jnp.int32, (nb, c), 0)
             == idsv[None, :])
        Of = O.astype(jnp.float32)

        cent_g = jax.lax.dot_general(cent_scr[...], Of,
                                     (((1,), (0,)), ((), ())),
                                     preferred_element_type=jnp.float32)
        rel = pos_b_ref[...] - cent_g
        d = jnp.sqrt(jnp.sum(rel * rel, axis=0, keepdims=True))
        rbfT = jnp.exp(-jnp.square(d - centers_ref[...]) * inv2w2_ref[...])
        geomT = (jnp.dot(WgT_ref[...], rbfT,
                         preferred_element_type=jnp.float32)
                 + bg_ref[...]).astype(jnp.bfloat16)
        geom_aug = jnp.concatenate(
            [geomT, jnp.ones((1, c), jnp.bfloat16)], axis=0)

        feats_f32 = feat_b_ref[...]
        feat_scr[pl.ds((i - nsteps) * c, c), :] = feats_f32
        feats = feats_f32.astype(jnp.bfloat16)

        S = (jax.lax.dot_general(qk1_scr[...], feats,
                                 (((1,), (1,)), ((), ())),
                                 preferred_element_type=jnp.float32)
             + jax.lax.dot_general(qk2_scr[...], geom_aug,
                                   (((1,), (0,)), ((), ())),
                                   preferred_element_type=jnp.float32))
        Sm = jnp.where(O, S, NEG)
        m_old = m_scr[...]
        m_new = jnp.maximum(m_old, jnp.max(Sm, axis=1, keepdims=True))
        alpha = jnp.exp(m_old - m_new)
        e = jnp.where(O, jnp.exp(S - m_new), 0.0)
        m_scr[...] = m_new
        s_scr[...] = s_scr[...] * alpha + jnp.sum(e, axis=1, keepdims=True)
        e_bf = e.astype(jnp.bfloat16)
        a1_scr[...] = (a1_scr[...] * alpha
                       + jnp.dot(e_bf, feats,
                                 preferred_element_type=jnp.float32))
        a2_scr[...] = (a2_scr[...] * alpha
                       + jax.lax.dot_general(e_bf, geom_aug,
                                             (((1,), (1,)), ((), ())),
                                             preferred_element_type=jnp.float32))

        @pl.when(i == 2 * nsteps - 1)
        def _fin_b():
            s = s_scr[...]
            craw = (jnp.dot(a1_scr[...], Wvt_ref[...].astype(jnp.float32),
                            preferred_element_type=jnp.float32)
                    + jnp.dot(a2_scr[...], Wvb_ref[...].astype(jnp.float32),
                              preferred_element_type=jnp.float32))
            ctx = craw / jnp.where(s > 0.0, s, 1.0)
            h1 = jnp.maximum(
                jnp.dot(ctx, Wc1_ref[...],
                        preferred_element_type=jnp.float32)
                + bc1_ref[...], 0.0)
            h_scr[...] = (jnp.dot(h1, Wc2_ref[...],
                                  preferred_element_type=jnp.float32)
                          + bc2_ref[...])

    @pl.when(i >= 2 * nsteps)
    def _phase_c():
        idsv = ids_c_ref[0, 0, :]
        Of = (jax.lax.broadcasted_iota(jnp.int32, (nb, c), 0)
              == idsv[None, :]).astype(jnp.float32)
        upd = jax.lax.dot_general(Of, h_scr[...], (((0,), (0,)), ((), ())),
                                  preferred_element_type=jnp.float32)
        feats = feat_scr[pl.ds((i - 2 * nsteps) * c, c), :]
        u1 = _ln_mxu(feats + upd, ln1g_ref[...], ln1b_ref[...], J_ref[...])
        f1 = jnp.maximum(
            jnp.dot(u1.astype(jnp.bfloat16), Wf1_ref[...],
                    preferred_element_type=jnp.float32)
            + bf1_ref[...], 0.0)
        ffn = (jnp.dot(f1.astype(jnp.bfloat16), Wf2_ref[...],
                       preferred_element_type=jnp.float32)
               + bf2_ref[...])
        out_ref[...] = _ln_mxu(u1 + ffn, ln2g_ref[...], ln2b_ref[...],
                               J_ref[...])


def kernel(atom_features, atom_positions, block_features, block_id,
           centers, widths, Wg, bg, Wq, bq, Wk, bk, Wv, bv,
           Wc1, bc1, Wc2, bc2, Wf1, bf1, Wf2, bf2,
           ln1_g, ln1_b, ln2_g, ln2_b):
    n, h = atom_features.shape
    nb = block_features.shape[0]
    rbf_dim = centers.shape[0]
    hq = Wg.shape[1]
    C = 8192
    nsteps = n // C

    ids = block_id.astype(jnp.int32)
    ids_chunked = ids.reshape(nsteps, 1, C)
    posT = atom_positions.T

    centers_col = centers.reshape(rbf_dim, 1).astype(jnp.float32)
    inv2w2_col = (1.0 / (2.0 * jnp.square(widths))).reshape(rbf_dim, 1)
    row = lambda v: v.reshape(1, -1)

    Wk_top = Wk[:h].astype(jnp.bfloat16)
    Wv_top = Wv[:h].astype(jnp.bfloat16)
    Wk_bot = jnp.concatenate([Wk[h:], bk.reshape(1, h)], 0).astype(jnp.bfloat16)
    Wv_bot = jnp.concatenate([Wv[h:], bv.reshape(1, h)], 0).astype(jnp.bfloat16)
    Jmat = jnp.full((h, h), 1.0 / h, jnp.float32)
    Wf1_bf = Wf1.astype(jnp.bfloat16)
    Wf2_bf = Wf2.astype(jnp.bfloat16)
    WgT = Wg.T
    bg_col = bg.reshape(hq, 1)

    last = nsteps - 1
    chunk_a = lambda i: (jnp.clip(i, 0, last), 0, 0)
    chunk_b = lambda i: (jnp.clip(i - nsteps, 0, last), 0, 0)
    chunk_c = lambda i: (jnp.clip(i - 2 * nsteps, 0, last), 0, 0)
    posm_a = lambda i: (0, jnp.clip(i, 0, last))
    posm_b = lambda i: (0, jnp.clip(i - nsteps, 0, last))
    featm_b = lambda i: (jnp.clip(i - nsteps, 0, last), 0)
    featm_c = lambda i: (jnp.clip(i - 2 * nsteps, 0, last), 0)
    full = lambda shape: pl.BlockSpec(shape, lambda i: (0,) * len(shape))

    out = pl.pallas_call(
        functools.partial(_fused_kernel, nb=nb, c=C, nsteps=nsteps),
        grid=(3 * nsteps,),
        in_specs=[
            pl.BlockSpec((1, 1, C), chunk_a),
            pl.BlockSpec((1, 1, C), chunk_b),
            pl.BlockSpec((1, 1, C), chunk_c),
            pl.BlockSpec((3, C), posm_a),
            pl.BlockSpec((3, C), posm_b),
            pl.BlockSpec((C, h), featm_b),
            full((nb, h)),
            full((rbf_dim, 1)),
            full((rbf_dim, 1)),
            full((hq, rbf_dim)),
            full((hq, 1)),
            full((h, h)),
            full((1, h)),
            full((h, h)),
            full((hq + 1, h)),
            full((h, h)),
            full((hq + 1, h)),
            full((h, h)),
            full((1, h)),
            full((h, h)),
            full((1, h)),
            full((h, 2 * h)),
            full((1, 2 * h)),
            full((2 * h, h)),
            full((1, h)),
            full((1, h)),
            full((1, h)),
            full((1, h)),
            full((1, h)),
            full((h, h)),
        ],
        out_specs=pl.BlockSpec((C, h), featm_c),
        out_shape=jax.ShapeDtypeStruct((n, h), jnp.float32),
        scratch_shapes=[
            pltpu.VMEM((3, nb), jnp.float32),
            pltpu.VMEM((1, nb), jnp.float32),
            pltpu.VMEM((3, nb), jnp.float32),
            pltpu.VMEM((nb, h), jnp.bfloat16),
            pltpu.VMEM((nb, hq + 1), jnp.bfloat16),
            pltpu.VMEM((nb, 1), jnp.float32),
            pltpu.VMEM((nb, 1), jnp.float32),
            pltpu.VMEM((nb, h), jnp.float32),
            pltpu.VMEM((nb, hq + 1), jnp.float32),
            pltpu.VMEM((nb, h), jnp.float32),
            pltpu.VMEM((n, h), jnp.float32),
        ],
    )(ids_chunked, ids_chunked, ids_chunked,
      posT, posT,
      atom_features, block_features,
      centers_col, inv2w2_col, WgT, bg_col,
      Wq, row(bq), Wk_top, Wk_bot,
      Wv_top, Wv_bot,
      Wc1, row(bc1), Wc2, row(bc2),
      Wf1_bf, row(bf1), Wf2_bf, row(bf2),
      row(ln1_g), row(ln1_b), row(ln2_g), row(ln2_b), Jmat)

    return out
